# baseline (speedup 1.0000x reference)
import functools

import jax
import jax.numpy as jnp
from jax import lax
from jax.experimental import pallas as pl
from jax.experimental.pallas import tpu as pltpu
from jax.experimental.pallas import tpu_sc as plsc

VOCAB = 100000
BATCH = 1024
SEQ = 200
LANES = 16
NUM_CORES = 2
NUM_SUBCORES = 16
NUM_WORKERS = NUM_CORES * NUM_SUBCORES
ROWS_PER_WORKER = BATCH // NUM_WORKERS
SEQ_PAD = 256
CHUNKS = SEQ_PAD // LANES
GROUP = 8
IDX_ROWS = 2 * GROUP
FLAT = BATCH * VOCAB

_ZBLOCK = 819200


def _zero_body(o_ref):
    o_ref[...] = jnp.zeros((_ZBLOCK,), jnp.float32)


_zero_fill = pl.pallas_call(
    _zero_body,
    out_shape=jax.ShapeDtypeStruct((FLAT,), jnp.float32),
    grid=(FLAT // _ZBLOCK,),
    out_specs=pl.BlockSpec((_ZBLOCK,), lambda i: (i,)),
)


def _scatter_body(tok_hbm, val_hbm, out_hbm, unused_out, tok_v, val_v,
                  row_buf, idx_st, cnt_st, sem):
    wid = lax.axis_index("s") * NUM_CORES + lax.axis_index("c")
    base = wid * ROWS_PER_WORKER

    zeros16 = jnp.zeros((LANES,), jnp.float32)

    def zero_body(i, carry):
        row_buf[pl.ds(i * LANES, LANES)] = zeros16
        return carry

    lax.fori_loop(0, VOCAB // LANES, zero_body, 0)

    def group_body(g, carry):
        grow = base + g * GROUP
        pltpu.sync_copy(tok_hbm.at[pl.ds(grow, GROUP)], tok_v)
        pltpu.sync_copy(val_hbm.at[pl.ds(grow, GROUP)], val_v)
        for r in range(GROUP):
            row = grow + r
            for c in range(CHUNKS):
                tok16 = tok_v[r, pl.ds(c * LANES, LANES)]
                v16 = val_v[r, pl.ds(c * LANES, LANES)]
                plsc.addupdate_scatter(row_buf, [tok16], v16)
            for c in range(CHUNKS):
                tok16 = tok_v[r, pl.ds(c * LANES, LANES)]
                cnt16 = plsc.load_gather(row_buf, [tok16])
                idx16 = tok16 * BATCH + row
                j = 2 * r + c // 8
                lane = (c % 8) * LANES
                idx_st[j, pl.ds(lane, LANES)] = idx16
                cnt_st[j, pl.ds(lane, LANES)] = cnt16
            for c in range(CHUNKS):
                tok16 = tok_v[r, pl.ds(c * LANES, LANES)]
                plsc.store_scatter(row_buf, [tok16], zeros16)
        copies = [
            pltpu.async_copy(cnt_st.at[j], out_hbm.at[idx_st.at[j]], sem)
            for j in range(IDX_ROWS)
        ]
        for cp in copies:
            cp.wait()
        return carry

    lax.fori_loop(0, ROWS_PER_WORKER // GROUP, group_body, 0)


_sc_scatter = functools.partial(
    pl.kernel,
    out_type=jax.ShapeDtypeStruct((8,), jnp.float32),
    mesh=plsc.VectorSubcoreMesh(core_axis_name="c", subcore_axis_name="s"),
    scratch_types=[
        pltpu.VMEM((GROUP, SEQ_PAD), jnp.int32),
        pltpu.VMEM((GROUP, SEQ_PAD), jnp.float32),
        pltpu.VMEM((VOCAB,), jnp.float32),
        pltpu.VMEM((IDX_ROWS, 128), jnp.int32),
        pltpu.VMEM((IDX_ROWS, 128), jnp.float32),
        pltpu.SemaphoreType.DMA,
    ],
    compiler_params=pltpu.CompilerParams(needs_layout_passes=False),
)(_scatter_body)


@jax.jit
def kernel(token_ids, mask):
    tok = jnp.pad(token_ids.astype(jnp.int32), ((0, 0), (0, SEQ_PAD - SEQ)))
    val = jnp.pad(mask.astype(jnp.float32), ((0, 0), (0, SEQ_PAD - SEQ)))
    out_ref = jax.new_ref(_zero_fill())
    _sc_scatter(tok, val, out_ref)
    return out_ref[...].reshape(VOCAB, BATCH).T


# speedup vs baseline: 3.0681x; 3.0681x over previous
import functools

import jax
import jax.numpy as jnp
from jax import lax
from jax.experimental import pallas as pl
from jax.experimental.pallas import tpu as pltpu
from jax.experimental.pallas import tpu_sc as plsc

VOCAB = 100000
BATCH = 1024
SEQ = 200
LANES = 16
NUM_CORES = 2
NUM_SUBCORES = 16
NUM_WORKERS = NUM_CORES * NUM_SUBCORES
ROWS_PER_WORKER = BATCH // NUM_WORKERS
SEQ_PAD = 208
CHUNKS = SEQ_PAD // LANES


def _count_body(tok_hbm, val_hbm, out_hbm, unused_out, tok_v, val_v, row_buf):
    wid = lax.axis_index("s") * NUM_CORES + lax.axis_index("c")
    base = wid * ROWS_PER_WORKER

    zeros16 = jnp.zeros((LANES,), jnp.float32)

    def zero_body(i, carry):
        row_buf[pl.ds(i * LANES, LANES)] = zeros16
        return carry

    lax.fori_loop(0, VOCAB // LANES, zero_body, 0)

    def row_body(r, carry):
        row = base + r
        pltpu.sync_copy(tok_hbm.at[row], tok_v)
        pltpu.sync_copy(val_hbm.at[row], val_v)
        for c in range(CHUNKS):
            idx = tok_v[pl.ds(c * LANES, LANES)]
            v = val_v[pl.ds(c * LANES, LANES)]
            plsc.addupdate_scatter(row_buf, [idx], v)
        pltpu.sync_copy(row_buf, out_hbm.at[row])
        for c in range(CHUNKS):
            idx = tok_v[pl.ds(c * LANES, LANES)]
            plsc.store_scatter(row_buf, [idx], zeros16)
        return carry

    lax.fori_loop(0, ROWS_PER_WORKER, row_body, 0)


_count_kernel = functools.partial(
    pl.kernel,
    out_type=jax.ShapeDtypeStruct((8,), jnp.float32),
    mesh=plsc.VectorSubcoreMesh(core_axis_name="c", subcore_axis_name="s"),
    scratch_types=[
        pltpu.VMEM((SEQ_PAD,), jnp.int32),
        pltpu.VMEM((SEQ_PAD,), jnp.float32),
        pltpu.VMEM((VOCAB,), jnp.float32),
    ],
    compiler_params=pltpu.CompilerParams(needs_layout_passes=False),
)(_count_body)


@jax.jit
def kernel(token_ids, mask):
    tok = jnp.pad(token_ids.astype(jnp.int32), ((0, 0), (0, SEQ_PAD - SEQ)))
    val = jnp.pad(mask.astype(jnp.float32), ((0, 0), (0, SEQ_PAD - SEQ)))
    out_ref = jax.empty_ref(
        jax.ShapeDtypeStruct((BATCH, VOCAB), jnp.float32))
    _count_kernel(tok, val, out_ref)
    return out_ref[...]
